# baseline (speedup 1.0000x reference)
import functools

import jax
import jax.numpy as jnp
from jax import lax
from jax.experimental import pallas as pl
from jax.experimental.pallas import tpu as pltpu
from jax.experimental.pallas import tpu_sc as plsc

NC = 2
NS = 16
NW = NC * NS
CH = 128
DP = 304


def _build(total, D, per_w, nch):
    mesh = plsc.VectorSubcoreMesh(core_axis_name="c", subcore_axis_name="s")
    nvr = DP // 16

    @functools.partial(
        pl.kernel,
        mesh=mesh,
        compiler_params=pltpu.CompilerParams(use_tc_tiling_on_sc=False),
        out_type=jax.ShapeDtypeStruct((total * D,), jnp.float32),
        scratch_types=[
            pltpu.VMEM((nch, CH), jnp.int32),
            pltpu.VMEM((CH, DP), jnp.float32),
            pltpu.VMEM((CH, DP), jnp.float32),
            pltpu.VMEM((CH * D + 16,), jnp.float32),
            pltpu.SemaphoreType.DMA,
            pltpu.SemaphoreType.DMA,
        ],
    )
    def k(x_hbm, table_hbm, out_hbm, idx_v, rows0, rows1, packed, sem0, sem1):
        wid = lax.axis_index("s") * NC + lax.axis_index("c")
        base = wid * per_w
        pltpu.sync_copy(x_hbm.at[wid], idx_v)

        bufs = ((rows0, sem0), (rows1, sem1))
        pltpu.async_copy(table_hbm.at[idx_v.at[0]], rows0, sem0)
        pltpu.async_copy(table_hbm.at[idx_v.at[1]], rows1, sem1)

        def body(jj, carry):
            for b, (rows, sem) in enumerate(bufs):
                j = jj + b
                pltpu.make_async_copy(table_hbm.at[idx_v.at[j]], rows, sem).wait()

                @plsc.parallel_loop(0, CH, unroll=4)
                def _rp_tail(r):
                    packed[pl.ds(r * D + 16 * (nvr - 1), 16)] = rows[
                        r, pl.ds(16 * (nvr - 1), 16)]

                @plsc.parallel_loop(0, CH, unroll=2)
                def _rp_body(r):
                    dst = r * D
                    for kk in range(nvr - 1):
                        packed[pl.ds(dst + 16 * kk, 16)] = rows[
                            r, pl.ds(16 * kk, 16)]

                jn = j + 2

                @pl.when(jn < nch)
                def _():
                    pltpu.async_copy(table_hbm.at[idx_v.at[jn]], rows, sem)

                pltpu.sync_copy(
                    packed.at[pl.ds(0, CH * D)],
                    out_hbm.at[pl.ds((base + j * CH) * D, CH * D)])
            return carry

        lax.fori_loop(0, nch // 2, lambda i, c: body(i * 2, c), 0)

    return k


CS = 240


def _build_t(B, CO):
    mesh = plsc.VectorSubcoreMesh(core_axis_name="c", subcore_axis_name="s")
    nblk = B // (128 * NW)
    ncs = CO // CS
    nu = nblk * ncs

    @functools.partial(
        pl.kernel,
        mesh=mesh,
        compiler_params=pltpu.CompilerParams(
            use_tc_tiling_on_sc=True, needs_layout_passes=False),
        out_type=jax.ShapeDtypeStruct((CO, B), jnp.float32),
        scratch_types=[
            pltpu.VMEM((128 * CS,), jnp.float32),
            pltpu.VMEM((128 * CS,), jnp.float32),
            pltpu.VMEM((CS, 128), jnp.float32),
            pltpu.VMEM((CS, 128), jnp.float32),
            pltpu.SemaphoreType.DMA,
            pltpu.SemaphoreType.DMA,
            pltpu.SemaphoreType.DMA,
            pltpu.SemaphoreType.DMA,
        ],
    )
    def k(flat_hbm, out_hbm, slab0, slab1, ob0, ob1, ls0, ls1, ss0, ss1):
        wid = lax.axis_index("s") * NC + lax.axis_index("c")
        blk0 = wid * nblk
        riota = lax.broadcasted_iota(jnp.int32, (16,), 0)
        riog = [(riota + 16 * g) * CS for g in range(8)]
        slabs = ((slab0, ls0), (slab1, ls1))
        obufs = ((ob0, ss0), (ob1, ss1))

        def fire(u, slab, sem):
            off = (blk0 + u // ncs) * 128 * CO + (u % ncs) * CS

            def one(i, c):
                pltpu.async_copy(flat_hbm.at[pl.ds(off + i * CO, CS)],
                                 slab.at[pl.ds(i * CS, CS)], sem)
                return c

            lax.fori_loop(0, 128, one, 0)

        def drain(slab, sem):
            def one(i, c):
                pltpu.make_async_copy(flat_hbm.at[pl.ds(0, CS)],
                                      slab.at[pl.ds(i * CS, CS)], sem).wait()
                return c

            lax.fori_loop(0, 128, one, 0)

        fire(0, slab0, ls0)

        def body(u, carry):
            for p in range(2):
                uu = u + p
                slab, lsem = slabs[p]
                obuf, ssem = obufs[p]

                @pl.when(uu + 1 < nu)
                def _():
                    fire(uu + 1, slabs[1 - p][0], slabs[1 - p][1])

                drain(slab, lsem)

                @pl.when(uu >= 2)
                def _():
                    pltpu.make_async_copy(
                        obuf, out_hbm.at[pl.ds(0, CS), pl.ds(0, 128)],
                        ssem).wait()

                @plsc.parallel_loop(0, CS, unroll=2)
                def _t(d):
                    for g in range(8):
                        v = plsc.load_gather(slab, [riog[g] + d])
                        obuf[d, pl.ds(16 * g, 16)] = v

                pltpu.async_copy(
                    obuf,
                    out_hbm.at[pl.ds((uu % ncs) * CS, CS),
                               pl.ds((blk0 + uu // ncs) * 128, 128)],
                    ssem)
            return carry

        lax.fori_loop(0, nu // 2, lambda i, c: body(i * 2, c), 0)
        for p in range(2):
            pltpu.make_async_copy(
                obufs[p][0], out_hbm.at[pl.ds(0, CS), pl.ds(0, 128)],
                obufs[p][1]).wait()

    return k


def _build_p(V, D, VP):
    mesh = plsc.VectorSubcoreMesh(core_axis_name="c", subcore_axis_name="s")
    nt = VP // 128
    no = DP // 8
    nfull = D // 8
    nvr = DP // 16

    @functools.partial(
        pl.kernel,
        mesh=mesh,
        compiler_params=pltpu.CompilerParams(
            use_tc_tiling_on_sc=True, needs_layout_passes=False),
        out_type=jax.ShapeDtypeStruct((VP * DP,), jnp.float32),
        scratch_types=[
            pltpu.VMEM((no, 8, 128), jnp.float32),
            pltpu.VMEM((no, 8, 128), jnp.float32),
            pltpu.VMEM((64 * DP,), jnp.float32),
            pltpu.VMEM((64 * DP,), jnp.float32),
            pltpu.SemaphoreType.DMA,
            pltpu.SemaphoreType.DMA,
            pltpu.SemaphoreType.DMA,
            pltpu.SemaphoreType.DMA,
        ],
    )
    def k(tt_hbm, out_hbm, sl0, sl1, pk0, pk1, l0, l1, s0, s1):
        wid = lax.axis_index("s") * NC + lax.axis_index("c")
        lo = wid * nt // NW
        hi = (wid + 1) * nt // NW
        lane = lax.broadcasted_iota(jnp.int32, (16,), 0)
        tvec = [2 * kk + lane // 8 for kk in range(nvr)]
        rvec = lane % 8
        slabs = ((sl0, l0), (sl1, l1))
        pks = ((pk0, s0), (pk1, s1))

        def fire(it, slab, sem):
            for t in range(nfull):
                pltpu.async_copy(
                    tt_hbm.at[pl.ds(8 * t, 8), pl.ds(128 * it, 128)],
                    slab.at[t], sem)
            pltpu.async_copy(
                tt_hbm.at[pl.ds(8 * nfull, D - 8 * nfull),
                          pl.ds(128 * it, 128)],
                slab.at[nfull, pl.ds(0, D - 8 * nfull)], sem)

        def drain(slab, sem):
            for t in range(nfull):
                pltpu.make_async_copy(
                    tt_hbm.at[pl.ds(0, 8), pl.ds(0, 128)],
                    slab.at[t], sem).wait()
            pltpu.make_async_copy(
                tt_hbm.at[pl.ds(0, D - 8 * nfull), pl.ds(0, 128)],
                slab.at[nfull, pl.ds(0, D - 8 * nfull)], sem).wait()

        fire(lo, sl0, l0)

        def body(j, carry):
            for p in range(2):
                it = j + p
                slab, lsem = slabs[p]

                @pl.when(it < hi)
                def _():
                    @pl.when(it + 1 < hi)
                    def _():
                        fire(it + 1, slabs[1 - p][0], slabs[1 - p][1])

                    drain(slab, lsem)

                    for h, (pk, ssem) in enumerate(pks):
                        @pl.when(it >= lo + 1)
                        def _():
                            pltpu.make_async_copy(
                                pk, out_hbm.at[pl.ds(0, 64 * DP)],
                                ssem).wait()

                        @plsc.parallel_loop(0, 64, unroll=2)
                        def _t(r):
                            cv = jnp.full((16,), 64 * h + r, jnp.int32)
                            for kk in range(nvr):
                                v = plsc.load_gather(
                                    slab, [tvec[kk], rvec, cv])
                                pk[pl.ds(r * DP + 16 * kk, 16)] = v

                        pltpu.async_copy(
                            pk,
                            out_hbm.at[pl.ds((it * 128 + 64 * h) * DP,
                                             64 * DP)], ssem)
            return carry

        nhalf = (nt + 2 * NW - 1) // (2 * NW) + 1
        lax.fori_loop(0, nhalf, lambda i, c: body(lo + i * 2, c), 0)
        for h, (pk, ssem) in enumerate(pks):
            pltpu.make_async_copy(
                pk, out_hbm.at[pl.ds(0, 64 * DP)], ssem).wait()

    return k


def kernel(x, table):
    B, S = x.shape
    V, D = table.shape
    total = B * S
    per_w = total // NW
    nch = per_w // CH
    VP = ((V + 127) // 128) * 128
    x_r = x.reshape(NW, nch, CH)
    tflat = _build_p(V, D, VP)(table.T)
    table_p = tflat.reshape(VP, DP)
    flat = _build(total, D, per_w, nch)(x_r, table_p)
    out_t = _build_t(B, S * D)(flat)
    return out_t.T


# speedup vs baseline: 1.1592x; 1.1592x over previous
import functools

import jax
import jax.numpy as jnp
from jax import lax
from jax.experimental import pallas as pl
from jax.experimental.pallas import tpu as pltpu
from jax.experimental.pallas import tpu_sc as plsc

NC = 2
NS = 16
NW = NC * NS
CH = 128
DP = 304


def _build(total, D, per_w, nch):
    mesh = plsc.VectorSubcoreMesh(core_axis_name="c", subcore_axis_name="s")
    nvr = DP // 16

    @functools.partial(
        pl.kernel,
        mesh=mesh,
        compiler_params=pltpu.CompilerParams(use_tc_tiling_on_sc=False),
        out_type=jax.ShapeDtypeStruct((total * D,), jnp.float32),
        scratch_types=[
            pltpu.VMEM((nch, CH), jnp.int32),
            pltpu.VMEM((CH, DP), jnp.float32),
            pltpu.VMEM((CH, DP), jnp.float32),
            pltpu.VMEM((CH * D + 16,), jnp.float32),
            pltpu.SemaphoreType.DMA,
            pltpu.SemaphoreType.DMA,
        ],
    )
    def k(x_hbm, table_hbm, out_hbm, idx_v, rows0, rows1, packed, sem0, sem1):
        wid = lax.axis_index("s") * NC + lax.axis_index("c")
        base = wid * per_w
        pltpu.sync_copy(x_hbm.at[wid], idx_v)

        bufs = ((rows0, sem0), (rows1, sem1))
        pltpu.async_copy(table_hbm.at[idx_v.at[0]], rows0, sem0)
        pltpu.async_copy(table_hbm.at[idx_v.at[1]], rows1, sem1)

        def body(jj, carry):
            for b, (rows, sem) in enumerate(bufs):
                j = jj + b
                pltpu.make_async_copy(table_hbm.at[idx_v.at[j]], rows, sem).wait()

                @plsc.parallel_loop(0, CH, unroll=4)
                def _rp_tail(r):
                    packed[pl.ds(r * D + 16 * (nvr - 1), 16)] = rows[
                        r, pl.ds(16 * (nvr - 1), 16)]

                @plsc.parallel_loop(0, CH, unroll=2)
                def _rp_body(r):
                    dst = r * D
                    for kk in range(nvr - 1):
                        packed[pl.ds(dst + 16 * kk, 16)] = rows[
                            r, pl.ds(16 * kk, 16)]

                jn = j + 2

                @pl.when(jn < nch)
                def _():
                    pltpu.async_copy(table_hbm.at[idx_v.at[jn]], rows, sem)

                pltpu.sync_copy(
                    packed.at[pl.ds(0, CH * D)],
                    out_hbm.at[pl.ds((base + j * CH) * D, CH * D)])
            return carry

        lax.fori_loop(0, nch // 2, lambda i, c: body(i * 2, c), 0)

    return k


CS = 200


def _build_t(B, CO):
    mesh = plsc.VectorSubcoreMesh(core_axis_name="c", subcore_axis_name="s")
    nblk = B // (128 * NW)
    ncs = CO // CS
    nu = nblk * ncs

    @functools.partial(
        pl.kernel,
        mesh=mesh,
        compiler_params=pltpu.CompilerParams(
            use_tc_tiling_on_sc=True, needs_layout_passes=False),
        out_type=jax.ShapeDtypeStruct((CO, B), jnp.float32),
        scratch_types=[
            pltpu.VMEM((128 * CS,), jnp.float32),
            pltpu.VMEM((128 * CS,), jnp.float32),
            pltpu.VMEM((CS, 128), jnp.float32),
            pltpu.VMEM((CS, 128), jnp.float32),
            pltpu.SemaphoreType.DMA,
            pltpu.SemaphoreType.DMA,
            pltpu.SemaphoreType.DMA,
            pltpu.SemaphoreType.DMA,
        ],
    )
    def k(flat_hbm, out_hbm, slab0, slab1, ob0, ob1, ls0, ls1, ss0, ss1):
        wid = lax.axis_index("s") * NC + lax.axis_index("c")
        blk0 = wid * nblk
        riota = lax.broadcasted_iota(jnp.int32, (16,), 0)
        riog = [(riota + 16 * g) * CS for g in range(8)]
        slabs = ((slab0, ls0), (slab1, ls1))
        obufs = ((ob0, ss0), (ob1, ss1))

        def fire(u, slab, sem):
            off = (blk0 + u // ncs) * 128 * CO + (u % ncs) * CS

            def one(i, c):
                pltpu.async_copy(flat_hbm.at[pl.ds(off + i * CO, CS)],
                                 slab.at[pl.ds(i * CS, CS)], sem)
                return c

            lax.fori_loop(0, 128, one, 0)

        def drain(slab, sem):
            def one(i, c):
                pltpu.make_async_copy(flat_hbm.at[pl.ds(0, CS)],
                                      slab.at[pl.ds(i * CS, CS)], sem).wait()
                return c

            lax.fori_loop(0, 128, one, 0)

        fire(0, slab0, ls0)

        def body(u, carry):
            for p in range(2):
                uu = u + p
                slab, lsem = slabs[p]
                obuf, ssem = obufs[p]

                @pl.when(uu + 1 < nu)
                def _():
                    fire(uu + 1, slabs[1 - p][0], slabs[1 - p][1])

                drain(slab, lsem)

                @pl.when(uu >= 2)
                def _():
                    pltpu.make_async_copy(
                        obuf, out_hbm.at[pl.ds(0, CS), pl.ds(0, 128)],
                        ssem).wait()

                @plsc.parallel_loop(0, CS, unroll=2)
                def _t(d):
                    for g in range(8):
                        v = plsc.load_gather(slab, [riog[g] + d])
                        obuf[d, pl.ds(16 * g, 16)] = v

                pltpu.async_copy(
                    obuf,
                    out_hbm.at[pl.ds((uu % ncs) * CS, CS),
                               pl.ds((blk0 + uu // ncs) * 128, 128)],
                    ssem)
            return carry

        lax.fori_loop(0, nu // 2, lambda i, c: body(i * 2, c), 0)
        for p in range(2):
            pltpu.make_async_copy(
                obufs[p][0], out_hbm.at[pl.ds(0, CS), pl.ds(0, 128)],
                obufs[p][1]).wait()

    return k


def _build_p(V, D, VP):
    mesh = plsc.VectorSubcoreMesh(core_axis_name="c", subcore_axis_name="s")
    nt = VP // 128
    no = DP // 8
    nfull = D // 8
    nvr = DP // 16

    @functools.partial(
        pl.kernel,
        mesh=mesh,
        compiler_params=pltpu.CompilerParams(
            use_tc_tiling_on_sc=True, needs_layout_passes=False),
        out_type=jax.ShapeDtypeStruct((VP * DP,), jnp.float32),
        scratch_types=[
            pltpu.VMEM((no, 8, 128), jnp.float32),
            pltpu.VMEM((no, 8, 128), jnp.float32),
            pltpu.VMEM((64 * DP,), jnp.float32),
            pltpu.VMEM((64 * DP,), jnp.float32),
            pltpu.SemaphoreType.DMA,
            pltpu.SemaphoreType.DMA,
            pltpu.SemaphoreType.DMA,
            pltpu.SemaphoreType.DMA,
        ],
    )
    def k(tt_hbm, out_hbm, sl0, sl1, pk0, pk1, l0, l1, s0, s1):
        wid = lax.axis_index("s") * NC + lax.axis_index("c")
        lo = wid * nt // NW
        hi = (wid + 1) * nt // NW
        lane = lax.broadcasted_iota(jnp.int32, (16,), 0)
        tvec = [2 * kk + lane // 8 for kk in range(nvr)]
        rvec = lane % 8
        slabs = ((sl0, l0), (sl1, l1))
        pks = ((pk0, s0), (pk1, s1))

        def fire(it, slab, sem):
            for t in range(nfull):
                pltpu.async_copy(
                    tt_hbm.at[pl.ds(8 * t, 8), pl.ds(128 * it, 128)],
                    slab.at[t], sem)
            pltpu.async_copy(
                tt_hbm.at[pl.ds(8 * nfull, D - 8 * nfull),
                          pl.ds(128 * it, 128)],
                slab.at[nfull, pl.ds(0, D - 8 * nfull)], sem)

        def drain(slab, sem):
            for t in range(nfull):
                pltpu.make_async_copy(
                    tt_hbm.at[pl.ds(0, 8), pl.ds(0, 128)],
                    slab.at[t], sem).wait()
            pltpu.make_async_copy(
                tt_hbm.at[pl.ds(0, D - 8 * nfull), pl.ds(0, 128)],
                slab.at[nfull, pl.ds(0, D - 8 * nfull)], sem).wait()

        fire(lo, sl0, l0)

        def body(j, carry):
            for p in range(2):
                it = j + p
                slab, lsem = slabs[p]

                @pl.when(it < hi)
                def _():
                    @pl.when(it + 1 < hi)
                    def _():
                        fire(it + 1, slabs[1 - p][0], slabs[1 - p][1])

                    drain(slab, lsem)

                    for h, (pk, ssem) in enumerate(pks):
                        @pl.when(it >= lo + 1)
                        def _():
                            pltpu.make_async_copy(
                                pk, out_hbm.at[pl.ds(0, 64 * DP)],
                                ssem).wait()

                        @plsc.parallel_loop(0, 64, unroll=2)
                        def _t(r):
                            cv = jnp.full((16,), 64 * h + r, jnp.int32)
                            for kk in range(nvr):
                                v = plsc.load_gather(
                                    slab, [tvec[kk], rvec, cv])
                                pk[pl.ds(r * DP + 16 * kk, 16)] = v

                        pltpu.async_copy(
                            pk,
                            out_hbm.at[pl.ds((it * 128 + 64 * h) * DP,
                                             64 * DP)], ssem)
            return carry

        nhalf = (nt + 2 * NW - 1) // (2 * NW) + 1
        lax.fori_loop(0, nhalf, lambda i, c: body(lo + i * 2, c), 0)
        for h, (pk, ssem) in enumerate(pks):
            pltpu.make_async_copy(
                pk, out_hbm.at[pl.ds(0, 64 * DP)], ssem).wait()

    return k


def kernel(x, table):
    B, S = x.shape
    V, D = table.shape
    total = B * S
    per_w = total // NW
    nch = per_w // CH
    VP = ((V + 127) // 128) * 128
    x_r = x.reshape(NW, nch, CH)
    tflat = _build_p(V, D, VP)(table.T)
    table_p = tflat.reshape(VP, DP)
    flat = _build(total, D, per_w, nch)(x_r, table_p)
    out_t = _build_t(B, S * D)(flat)
    return out_t.T


# speedup vs baseline: 1.1654x; 1.0053x over previous
import functools

import jax
import jax.numpy as jnp
from jax import lax
from jax.experimental import pallas as pl
from jax.experimental.pallas import tpu as pltpu
from jax.experimental.pallas import tpu_sc as plsc

NC = 2
NS = 16
NW = NC * NS
CH = 128
DP = 304


def _build(total, D, per_w, nch):
    mesh = plsc.VectorSubcoreMesh(core_axis_name="c", subcore_axis_name="s")
    nvr = DP // 16

    @functools.partial(
        pl.kernel,
        mesh=mesh,
        compiler_params=pltpu.CompilerParams(use_tc_tiling_on_sc=False),
        out_type=jax.ShapeDtypeStruct((total * D,), jnp.float32),
        scratch_types=[
            pltpu.VMEM((nch, CH), jnp.int32),
            pltpu.VMEM((CH, DP), jnp.float32),
            pltpu.VMEM((CH, DP), jnp.float32),
            pltpu.VMEM((CH * D + 16,), jnp.float32),
            pltpu.SemaphoreType.DMA,
            pltpu.SemaphoreType.DMA,
        ],
    )
    def k(x_hbm, table_hbm, out_hbm, idx_v, rows0, rows1, packed, sem0, sem1):
        wid = lax.axis_index("s") * NC + lax.axis_index("c")
        base = wid * per_w
        pltpu.sync_copy(x_hbm.at[wid], idx_v)

        bufs = ((rows0, sem0), (rows1, sem1))
        pltpu.async_copy(table_hbm.at[idx_v.at[0]], rows0, sem0)
        pltpu.async_copy(table_hbm.at[idx_v.at[1]], rows1, sem1)

        def body(jj, carry):
            for b, (rows, sem) in enumerate(bufs):
                j = jj + b
                pltpu.make_async_copy(table_hbm.at[idx_v.at[j]], rows, sem).wait()

                @plsc.parallel_loop(0, CH, unroll=4)
                def _rp_tail(r):
                    packed[pl.ds(r * D + 16 * (nvr - 1), 16)] = rows[
                        r, pl.ds(16 * (nvr - 1), 16)]

                @plsc.parallel_loop(0, CH, unroll=2)
                def _rp_body(r):
                    dst = r * D
                    for kk in range(nvr - 1):
                        packed[pl.ds(dst + 16 * kk, 16)] = rows[
                            r, pl.ds(16 * kk, 16)]

                jn = j + 2

                @pl.when(jn < nch)
                def _():
                    pltpu.async_copy(table_hbm.at[idx_v.at[jn]], rows, sem)

                pltpu.sync_copy(
                    packed.at[pl.ds(0, CH * D)],
                    out_hbm.at[pl.ds((base + j * CH) * D, CH * D)])
            return carry

        lax.fori_loop(0, nch // 2, lambda i, c: body(i * 2, c), 0)

    return k


CS = 200


def _build_t(B, CO):
    mesh = plsc.VectorSubcoreMesh(core_axis_name="c", subcore_axis_name="s")
    nblk = B // (128 * NW)
    ncs = CO // CS
    nu = nblk * ncs

    @functools.partial(
        pl.kernel,
        mesh=mesh,
        compiler_params=pltpu.CompilerParams(
            use_tc_tiling_on_sc=True, needs_layout_passes=False),
        out_type=jax.ShapeDtypeStruct((CO, B), jnp.float32),
        scratch_types=[
            pltpu.VMEM((128 * CS,), jnp.float32),
            pltpu.VMEM((128 * CS,), jnp.float32),
            pltpu.VMEM((CS, 128), jnp.float32),
            pltpu.VMEM((CS, 128), jnp.float32),
            pltpu.SemaphoreType.DMA,
            pltpu.SemaphoreType.DMA,
            pltpu.SemaphoreType.DMA,
            pltpu.SemaphoreType.DMA,
        ],
    )
    def k(flat_hbm, out_hbm, slab0, slab1, ob0, ob1, ls0, ls1, ss0, ss1):
        wid = lax.axis_index("s") * NC + lax.axis_index("c")
        blk0 = wid * nblk
        riota = lax.broadcasted_iota(jnp.int32, (16,), 0)
        riog = [(riota + 16 * g) * CS for g in range(8)]
        slabs = ((slab0, ls0), (slab1, ls1))
        obufs = ((ob0, ss0), (ob1, ss1))

        def fire(u, slab, sem):
            off = (blk0 + u // ncs) * 128 * CO + (u % ncs) * CS

            def one(i, c):
                pltpu.async_copy(flat_hbm.at[pl.ds(off + i * CO, CS)],
                                 slab.at[pl.ds(i * CS, CS)], sem)
                return c

            lax.fori_loop(0, 128, one, 0)

        def drain(slab, sem):
            def one(i, c):
                pltpu.make_async_copy(flat_hbm.at[pl.ds(0, CS)],
                                      slab.at[pl.ds(i * CS, CS)], sem).wait()
                return c

            lax.fori_loop(0, 128, one, 0)

        fire(0, slab0, ls0)

        def body(u, carry):
            for p in range(2):
                uu = u + p
                slab, lsem = slabs[p]
                obuf, ssem = obufs[p]

                @pl.when(uu + 1 < nu)
                def _():
                    fire(uu + 1, slabs[1 - p][0], slabs[1 - p][1])

                drain(slab, lsem)

                @pl.when(uu >= 2)
                def _():
                    pltpu.make_async_copy(
                        obuf, out_hbm.at[pl.ds(0, CS), pl.ds(0, 128)],
                        ssem).wait()

                @plsc.parallel_loop(0, CS, unroll=4)
                def _t(d):
                    for g in range(8):
                        v = plsc.load_gather(slab, [riog[g] + d])
                        obuf[d, pl.ds(16 * g, 16)] = v

                pltpu.async_copy(
                    obuf,
                    out_hbm.at[pl.ds((uu % ncs) * CS, CS),
                               pl.ds((blk0 + uu // ncs) * 128, 128)],
                    ssem)
            return carry

        lax.fori_loop(0, nu // 2, lambda i, c: body(i * 2, c), 0)
        for p in range(2):
            pltpu.make_async_copy(
                obufs[p][0], out_hbm.at[pl.ds(0, CS), pl.ds(0, 128)],
                obufs[p][1]).wait()

    return k


def _build_p(V, D, VP):
    mesh = plsc.VectorSubcoreMesh(core_axis_name="c", subcore_axis_name="s")
    nt = VP // 128
    no = DP // 8
    nfull = D // 8
    nvr = DP // 16

    @functools.partial(
        pl.kernel,
        mesh=mesh,
        compiler_params=pltpu.CompilerParams(
            use_tc_tiling_on_sc=True, needs_layout_passes=False),
        out_type=jax.ShapeDtypeStruct((VP * DP,), jnp.float32),
        scratch_types=[
            pltpu.VMEM((no, 8, 128), jnp.float32),
            pltpu.VMEM((no, 8, 128), jnp.float32),
            pltpu.VMEM((64 * DP,), jnp.float32),
            pltpu.VMEM((64 * DP,), jnp.float32),
            pltpu.SemaphoreType.DMA,
            pltpu.SemaphoreType.DMA,
            pltpu.SemaphoreType.DMA,
            pltpu.SemaphoreType.DMA,
        ],
    )
    def k(tt_hbm, out_hbm, sl0, sl1, pk0, pk1, l0, l1, s0, s1):
        wid = lax.axis_index("s") * NC + lax.axis_index("c")
        lo = wid * nt // NW
        hi = (wid + 1) * nt // NW
        lane = lax.broadcasted_iota(jnp.int32, (16,), 0)
        tvec = [2 * kk + lane // 8 for kk in range(nvr)]
        rvec = lane % 8
        slabs = ((sl0, l0), (sl1, l1))
        pks = ((pk0, s0), (pk1, s1))

        def fire(it, slab, sem):
            for t in range(nfull):
                pltpu.async_copy(
                    tt_hbm.at[pl.ds(8 * t, 8), pl.ds(128 * it, 128)],
                    slab.at[t], sem)
            pltpu.async_copy(
                tt_hbm.at[pl.ds(8 * nfull, D - 8 * nfull),
                          pl.ds(128 * it, 128)],
                slab.at[nfull, pl.ds(0, D - 8 * nfull)], sem)

        def drain(slab, sem):
            for t in range(nfull):
                pltpu.make_async_copy(
                    tt_hbm.at[pl.ds(0, 8), pl.ds(0, 128)],
                    slab.at[t], sem).wait()
            pltpu.make_async_copy(
                tt_hbm.at[pl.ds(0, D - 8 * nfull), pl.ds(0, 128)],
                slab.at[nfull, pl.ds(0, D - 8 * nfull)], sem).wait()

        fire(lo, sl0, l0)

        def body(j, carry):
            for p in range(2):
                it = j + p
                slab, lsem = slabs[p]

                @pl.when(it < hi)
                def _():
                    @pl.when(it + 1 < hi)
                    def _():
                        fire(it + 1, slabs[1 - p][0], slabs[1 - p][1])

                    drain(slab, lsem)

                    for h, (pk, ssem) in enumerate(pks):
                        @pl.when(it >= lo + 1)
                        def _():
                            pltpu.make_async_copy(
                                pk, out_hbm.at[pl.ds(0, 64 * DP)],
                                ssem).wait()

                        @plsc.parallel_loop(0, 64, unroll=4)
                        def _t(r):
                            cv = jnp.full((16,), 64 * h + r, jnp.int32)
                            for kk in range(nvr):
                                v = plsc.load_gather(
                                    slab, [tvec[kk], rvec, cv])
                                pk[pl.ds(r * DP + 16 * kk, 16)] = v

                        pltpu.async_copy(
                            pk,
                            out_hbm.at[pl.ds((it * 128 + 64 * h) * DP,
                                             64 * DP)], ssem)
            return carry

        nhalf = (nt + 2 * NW - 1) // (2 * NW) + 1
        lax.fori_loop(0, nhalf, lambda i, c: body(lo + i * 2, c), 0)
        for h, (pk, ssem) in enumerate(pks):
            pltpu.make_async_copy(
                pk, out_hbm.at[pl.ds(0, 64 * DP)], ssem).wait()

    return k


def kernel(x, table):
    B, S = x.shape
    V, D = table.shape
    total = B * S
    per_w = total // NW
    nch = per_w // CH
    VP = ((V + 127) // 128) * 128
    x_r = x.reshape(NW, nch, CH)
    tflat = _build_p(V, D, VP)(table.T)
    table_p = tflat.reshape(VP, DP)
    flat = _build(total, D, per_w, nch)(x_r, table_p)
    out_t = _build_t(B, S * D)(flat)
    return out_t.T
